# baseline (speedup 1.0000x reference)
import functools

import jax
import jax.numpy as jnp
import numpy as np
from jax import lax
from jax.experimental import pallas as pl
from jax.experimental.pallas import tpu as pltpu
from jax.experimental.pallas import tpu_sc as plsc

N_NODES = 10000
ATOM_IN = 92
EDGE_BINS = 80
TRIPLET_BINS = 40
EMBED = 64
HIDDEN = 128
R_ONSET = 7.5
R_CUTOFF = 8.0

LANES = 128
BN = 1024


def _mm_fwd_kernel(x_ref, w_ref, b_ref, o_ref):
    o_ref[...] = (
        jnp.dot(x_ref[...], w_ref[...], preferred_element_type=jnp.float32)
        + b_ref[...]
    )


def _mm_nt_kernel(g_ref, w_ref, o_ref):
    o_ref[...] = lax.dot_general(
        g_ref[...], w_ref[...], (((1,), (1,)), ((), ())),
        preferred_element_type=jnp.float32)


def _mm_tn_kernel(x_ref, g_ref, dw_ref, db_ref, *, nrows):
    i = pl.program_id(0)
    rows = lax.broadcasted_iota(jnp.int32, (BN, 1), 0) + i * BN
    keep = rows < nrows
    x = jnp.where(keep, x_ref[...], 0.0)
    g = jnp.where(keep, g_ref[...], 0.0)

    @pl.when(i == 0)
    def _():
        dw_ref[...] = jnp.zeros_like(dw_ref)
        db_ref[...] = jnp.zeros_like(db_ref)

    dw_ref[...] += lax.dot_general(
        x, g, (((0,), (0,)), ((), ())), preferred_element_type=jnp.float32)
    db_ref[...] += jnp.sum(g, axis=0, keepdims=True)


def _mm_raw(x, w, b):
    n = x.shape[0]
    return pl.pallas_call(
        _mm_fwd_kernel,
        grid=(pl.cdiv(n, BN),),
        in_specs=[
            pl.BlockSpec((BN, LANES), lambda i: (i, 0)),
            pl.BlockSpec((LANES, LANES), lambda i: (0, 0)),
            pl.BlockSpec((1, LANES), lambda i: (0, 0)),
        ],
        out_specs=pl.BlockSpec((BN, LANES), lambda i: (i, 0)),
        out_shape=jax.ShapeDtypeStruct((n, LANES), jnp.float32),
    )(x, w, b.reshape(1, LANES))


def _mm_nt_raw(g, w):
    n = g.shape[0]
    return pl.pallas_call(
        _mm_nt_kernel,
        grid=(pl.cdiv(n, BN),),
        in_specs=[
            pl.BlockSpec((BN, LANES), lambda i: (i, 0)),
            pl.BlockSpec((LANES, LANES), lambda i: (0, 0)),
        ],
        out_specs=pl.BlockSpec((BN, LANES), lambda i: (i, 0)),
        out_shape=jax.ShapeDtypeStruct((n, LANES), jnp.float32),
    )(g, w)


def _mm_tn_raw(x, g):
    n = x.shape[0]
    dw, db = pl.pallas_call(
        functools.partial(_mm_tn_kernel, nrows=n),
        grid=(pl.cdiv(n, BN),),
        in_specs=[
            pl.BlockSpec((BN, LANES), lambda i: (i, 0)),
            pl.BlockSpec((BN, LANES), lambda i: (i, 0)),
        ],
        out_specs=[
            pl.BlockSpec((LANES, LANES), lambda i: (0, 0)),
            pl.BlockSpec((1, LANES), lambda i: (0, 0)),
        ],
        out_shape=[
            jax.ShapeDtypeStruct((LANES, LANES), jnp.float32),
            jax.ShapeDtypeStruct((1, LANES), jnp.float32),
        ],
    )(x, g)
    return dw, db.reshape(LANES)


@jax.custom_vjp
def mm(x, w, b):
    return _mm_raw(x, w, b)


def _mm_vfwd(x, w, b):
    return _mm_raw(x, w, b), (x, w)


def _mm_vbwd(res, g):
    x, w = res
    dx = _mm_nt_raw(g, w)
    dw, db = _mm_tn_raw(x, g)
    return dx, dw, db


mm.defvjp(_mm_vfwd, _mm_vbwd)


CH = 128
NW = 32


POOL_WORDS = 2040000


def _sc_gather_raw(table, idx):
    t_rows, d = table.shape
    b = idx.shape[0]
    per = b // NW
    nfull, tail = per // CH, per % CH
    GG = 4 if d >= 64 else 8
    ngrp, nrem = nfull // GG, nfull % GG
    mesh = plsc.VectorSubcoreMesh(core_axis_name="c", subcore_axis_name="s")

    @functools.partial(
        pl.kernel,
        out_type=jax.ShapeDtypeStruct((b, d), jnp.float32),
        mesh=mesh,
        compiler_params=pltpu.CompilerParams(use_tc_tiling_on_sc=False),
        scratch_types=[
            pltpu.VMEM((GG, CH), jnp.int32),
            pltpu.VMEM((GG, CH, d), jnp.float32),
            pltpu.VMEM((max(tail, 8),), jnp.int32),
            pltpu.VMEM((max(tail, 8), d), jnp.float32),
            pltpu.SemaphoreType.DMA,
            pltpu.SemaphoreType.DMA,
            pltpu.SemaphoreType.DMA,
        ],
    )
    def k(table_hbm, idx_hbm, out_hbm, idx_v, rows_v, idxt_v, rowst_v,
          sem_i, sem_g, sem_o):
        w = lax.axis_index("s") * 2 + lax.axis_index("c")
        base = w * per

        def group(g, nch):
            off0 = base + g * (GG * CH)
            hs = [pltpu.async_copy(
                idx_hbm.at[pl.ds(off0 + q * CH, CH)], idx_v.at[q], sem_i)
                for q in range(nch)]
            for h in hs:
                h.wait()
            hs = [pltpu.async_copy(
                table_hbm.at[idx_v.at[q]], rows_v.at[q], sem_g)
                for q in range(nch)]
            for h in hs:
                h.wait()
            hs = [pltpu.async_copy(
                rows_v.at[q], out_hbm.at[pl.ds(off0 + q * CH, CH)], sem_o)
                for q in range(nch)]
            for h in hs:
                h.wait()

        def body(g, _):
            group(g, GG)
            return 0

        lax.fori_loop(0, ngrp, body, 0)
        if nrem:
            group(ngrp, nrem)
        if tail:
            off = base + nfull * CH
            pltpu.sync_copy(idx_hbm.at[pl.ds(off, tail)], idxt_v)
            pltpu.async_copy(table_hbm.at[idxt_v], rowst_v, sem_g).wait()
            pltpu.sync_copy(rowst_v, out_hbm.at[pl.ds(off, tail)])

    return k(table, idx)


def _scatter_gg(d):
    return 1 if d > 64 else 8


def _sc_scatter_raw(vals, idx, v_pad):
    b, d = vals.shape
    per = b // NW
    nfull, tail = per // CH, per % CH
    dc = d
    GG = _scatter_gg(d)
    ngrp, nrem = nfull // GG, nfull % GG
    nd = 1
    rpt = v_pad // 16
    zr = min(rpt, 64 if d >= 128 else 128)
    nz, ztail = rpt // zr, rpt % zr
    mesh = plsc.VectorSubcoreMesh(core_axis_name="c", subcore_axis_name="s")

    @functools.partial(
        pl.kernel,
        out_type=jax.ShapeDtypeStruct((2, v_pad, d), jnp.float32),
        mesh=mesh,
        compiler_params=pltpu.CompilerParams(
            use_tc_tiling_on_sc=False, has_side_effects=True),
        scratch_types=[
            pltpu.VMEM((GG, CH), jnp.int32),
            pltpu.VMEM((GG, CH, dc), jnp.float32),
            pltpu.VMEM((max(tail, 8),), jnp.int32),
            pltpu.VMEM((max(tail, 8), dc), jnp.float32),
            pltpu.VMEM((zr, dc), jnp.float32),
            pltpu.VMEM_SHARED((v_pad, dc), jnp.float32),
            pltpu.SemaphoreType.DMA,
            pltpu.SemaphoreType.DMA,
            pltpu.SemaphoreType.DMA,
        ],
    )
    def k(vals_hbm, idx_hbm, zeros_hbm, out_hbm, idx_v, vals_v, idxt_v,
          valst_v, z_v, shared, sem_i, sem_v, sem_a):
        c = lax.axis_index("c")
        s = lax.axis_index("s")
        w = s * 2 + c
        row0 = s * rpt
        base = w * per
        pltpu.sync_copy(zeros_hbm, z_v)

        def group(g, nch, c0):
            off0 = base + g * (GG * CH)
            hs = [pltpu.async_copy(
                idx_hbm.at[pl.ds(off0 + q * CH, CH)], idx_v.at[q], sem_i)
                for q in range(nch)]
            hs += [pltpu.async_copy(
                vals_hbm.at[pl.ds(off0 + q * CH, CH), pl.ds(c0, dc)],
                vals_v.at[q], sem_v) for q in range(nch)]
            for h in hs:
                h.wait()
            hs = [pltpu.async_copy(
                vals_v.at[q], shared.at[idx_v.at[q]], sem_a, add=True)
                for q in range(nch)]
            for h in hs:
                h.wait()

        for di in range(nd):
            c0 = di * dc

            def zbody(j, _):
                pltpu.sync_copy(z_v, shared.at[pl.ds(row0 + j * zr, zr)])
                return 0

            lax.fori_loop(0, nz, zbody, 0)
            if ztail:
                pltpu.sync_copy(z_v.at[pl.ds(0, ztail)],
                                shared.at[pl.ds(row0 + nz * zr, ztail)])
            plsc.subcore_barrier()

            def body(g, _):
                group(g, GG, c0)
                return 0

            lax.fori_loop(0, ngrp, body, 0)
            if nrem:
                group(ngrp, nrem, c0)
            if tail:
                off = base + nfull * CH
                pltpu.sync_copy(idx_hbm.at[pl.ds(off, tail)], idxt_v)
                pltpu.sync_copy(
                    vals_hbm.at[pl.ds(off, tail), pl.ds(c0, dc)], valst_v)
                pltpu.sync_copy(valst_v, shared.at[idxt_v], add=True)
            plsc.subcore_barrier()
            pltpu.sync_copy(
                shared.at[pl.ds(row0, rpt)],
                out_hbm.at[c, pl.ds(row0, rpt), pl.ds(c0, dc)])
            if nd > 1 and di < nd - 1:
                plsc.subcore_barrier()

    return k(vals, idx, jnp.zeros((zr, dc), jnp.float32))


def _round_up(n, m):
    return ((n + m - 1) // m) * m


def _pad_rows(x, n_pad, value=0.0):
    n = x.shape[0]
    if n == n_pad:
        return x
    return jnp.pad(x, ((0, n_pad - n),) + ((0, 0),) * (x.ndim - 1),
                   constant_values=value)


def _scatter_any(vals, idx, v_rows):
    d = vals.shape[1]
    cap = 12544 if d >= 128 else 102400
    pieces = []
    for lo in range(0, v_rows, cap):
        rows = min(cap, v_rows - lo)
        v_pad = _round_up(rows + 1, 128)
        local = idx - lo
        ok = (local >= 0) & (local < rows)
        local = jnp.where(ok, local, rows).astype(jnp.int32)
        part = _sc_scatter_raw(vals, local, v_pad)
        pieces.append((part[0] + part[1])[:rows])
    return jnp.concatenate(pieces, 0) if len(pieces) > 1 else pieces[0]


def _gather_any(table, idx):
    ok = (idx >= 0) & (idx < table.shape[0])
    safe = jnp.where(ok, idx, 0).astype(jnp.int32)
    out = _sc_gather_raw(table, safe)
    return out * ok[:, None].astype(jnp.float32)


@functools.partial(jax.custom_vjp, nondiff_argnums=())
def sc_gather(table, idx):
    return _gather_any(table, idx)


def _sc_gather_vfwd(table, idx):
    return _gather_any(table, idx), (idx, table.shape[0])


def _sc_gather_vbwd(res, g):
    idx, t_rows = res
    return _scatter_any(g, idx, t_rows), None


sc_gather.defvjp(_sc_gather_vfwd, _sc_gather_vbwd)


@jax.custom_vjp
def sc_gather_pair(table, idx, bwd_idx):
    return _gather_any(table, idx)


def _sc_gather_pair_vfwd(table, idx, bwd_idx):
    return _gather_any(table, idx), (bwd_idx,)


def _sc_gather_pair_vbwd(res, g):
    (bwd_idx,) = res
    return _gather_any(g, bwd_idx), None, None


sc_gather_pair.defvjp(_sc_gather_pair_vfwd, _sc_gather_pair_vbwd)


@functools.partial(jax.custom_vjp, nondiff_argnums=(2,))
def sc_segment_sum(vals, idx, v_rows):
    return _scatter_any(vals, idx, v_rows)


def _sc_segsum_vfwd(vals, idx, v_rows):
    return _scatter_any(vals, idx, v_rows), (idx,)


def _sc_segsum_vbwd(v_rows, res, g):
    (idx,) = res
    return _gather_any(g, idx), None


sc_segment_sum.defvjp(_sc_segsum_vfwd, _sc_segsum_vbwd)


def _pad_lin(p, din, dout):
    w = jnp.pad(p["W"], ((0, LANES - din), (0, LANES - dout)))
    b = jnp.pad(p["b"], (0, LANES - dout))
    return w, b


def _silu(x):
    return x * jax.nn.sigmoid(x)


def _masked_ln_silu(u, g, be, dtrue):
    if dtrue == LANES:
        mu = jnp.mean(u, -1, keepdims=True)
        dev = u - mu
        var = jnp.mean(dev * dev, -1, keepdims=True)
        return _silu(dev * lax.rsqrt(var + 1e-5) * g + be)
    colmask = (jnp.arange(LANES) < dtrue).astype(jnp.float32)
    mu = jnp.sum(u, -1, keepdims=True) / dtrue
    dev = (u - mu) * colmask
    var = jnp.sum(dev * dev, -1, keepdims=True) / dtrue
    return _silu(dev * lax.rsqrt(var + 1e-5) * g + be) * colmask


def _mlp(pp, x, dtrue):
    w, b, g, be = pp
    return _masked_ln_silu(mm(x, w, b), g, be, dtrue)


def _prep_mlp(p, din, dout):
    w, b = _pad_lin(p["lin"], din, dout)
    g = jnp.pad(p["g"], (0, LANES - dout))
    be = jnp.pad(p["be"], (0, LANES - dout))
    return (w, b, g, be)


def _rbf(d, vmin, vmax, bins):
    centers = jnp.linspace(vmin, vmax, bins)
    gamma = 1.0 / (centers[1] - centers[0])
    centers = jnp.concatenate(
        [centers, jnp.full((LANES - bins,), 1e9, jnp.float32)])
    return jnp.exp(-gamma * (d[:, None] - centers[None, :]) ** 2)


def _smooth_cutoff(r):
    rc2, ro2, r2 = R_CUTOFF**2, R_ONSET**2, r**2
    fc = ((rc2 - r2) ** 2 * (rc2 + 2.0 * r2 - 3.0 * ro2)) / (rc2 - ro2) ** 3
    return jnp.where(r < R_ONSET, 1.0, jnp.where(r > R_CUTOFF, 0.0, fc))


def _egc(p, src_i, dst_i, x, y, v_rows, cutoff=None):
    wsg, bsg = _pad_lin(p["src_gate"], HIDDEN, HIDDEN)
    wdg, bdg = _pad_lin(p["dst_gate"], HIDDEN, HIDDEN)
    weg, beg = _pad_lin(p["edge_gate"], HIDDEN, HIDDEN)
    wsu, bsu = _pad_lin(p["src_update"], HIDDEN, HIDDEN)
    wdu, bdu = _pad_lin(p["dst_update"], HIDDEN, HIDDEN)
    gxs = mm(x, wsg, bsg)
    gxd = mm(x, wdg, bdg)
    ey = mm(y, weg, beg)
    m = sc_gather(gxs, src_i) + sc_gather(gxd, dst_i) + ey
    sigma = jax.nn.sigmoid(m)
    if cutoff is not None:
        sigma = sigma * cutoff[:, None]
    bh = mm(x, wdu, bdu)
    sbh = sc_gather(bh, src_i)
    num = sc_segment_sum(sigma * sbh, dst_i, v_rows)
    den = sc_segment_sum(sigma, dst_i, v_rows)
    h = num / (den + 1e-6)
    x_out = x + _masked_ln_silu(mm(x, wsu, bsu) + h, p["ng"], p["nb"], HIDDEN)
    y_out = y + _masked_ln_silu(m, p["eg"], p["eb"], HIDDEN)
    return x_out, y_out


def kernel(atom_features, r, edge_index, y_mask, lg_src, lg_dst, params):
    src = edge_index[0]
    dst = edge_index[1]
    n_edges = src.shape[0]
    n_local = y_mask.shape[0]
    n_lg = lg_src.shape[0]

    n_local_p = _round_up(n_local + 1, 256)
    n_lg_p = _round_up(n_lg, 256)

    y_mask_p = _pad_rows(y_mask, n_local_p, -1)
    lg_src_p = _pad_rows(lg_src, n_lg_p, -1)
    lg_dst_p = _pad_rows(lg_dst, n_lg_p, -1)

    inv = jnp.full((n_edges,), -1, jnp.int32)
    inv = inv.at[y_mask].set(jnp.arange(n_local, dtype=jnp.int32))

    af_p = jnp.pad(atom_features, ((0, 0), (0, LANES - ATOM_IN)))
    r16 = jnp.pad(r, ((0, 0), (0, 13)))

    edge_mlp1 = _prep_mlp(params["edge_mlp1"], EDGE_BINS, EMBED)
    edge_mlp2 = _prep_mlp(params["edge_mlp2"], EMBED, HIDDEN)
    angle_mlp1 = _prep_mlp(params["angle_mlp1"], TRIPLET_BINS, EMBED)
    angle_mlp2 = _prep_mlp(params["angle_mlp2"], EMBED, HIDDEN)
    atom_mlp = _prep_mlp(params["atom_mlp"], ATOM_IN, HIDDEN)
    wfc, bfc = _pad_lin(params["fc"], HIDDEN, 1)

    def energy(r16_in):
        bondlength = jnp.sqrt(jnp.sum(r16_in * r16_in, 1))
        fcut = _smooth_cutoff(bondlength)
        y = _mlp(edge_mlp2,
                 _mlp(edge_mlp1, _rbf(bondlength, 0.0, 8.0, EDGE_BINS),
                      EMBED), HIDDEN)
        r_local = sc_gather_pair(r16_in, y_mask_p, inv)
        r1 = -sc_gather(r_local, lg_src_p)
        r2 = sc_gather(r_local, lg_dst_p)
        dotp = jnp.sum(r1 * r2, 1)
        nrm = jnp.sqrt(jnp.sum(r1 * r1, 1) * jnp.sum(r2 * r2, 1))
        cos = jnp.clip(dotp / jnp.maximum(nrm, 1e-30), -1.0, 1.0)
        z = _mlp(angle_mlp2,
                 _mlp(angle_mlp1, _rbf(cos, -1.0, 1.0, TRIPLET_BINS),
                      EMBED), HIDDEN)
        x = _mlp(atom_mlp, af_p, HIDDEN)
        for lp in params["alignn"]:
            ylocal = sc_gather_pair(y, y_mask_p, inv)
            m_e, z = _egc(lp["edge_update"], lg_src_p, lg_dst_p, ylocal, z,
                          n_local_p)
            y = y + sc_gather_pair(m_e - ylocal, inv, y_mask_p)
            x, y = _egc(lp["node_update"], src, dst, x, y, N_NODES, fcut)
        for lp in params["gcn"]:
            x, y = _egc(lp, src, dst, x, y, N_NODES, fcut)
        atomwise = mm(x, wfc, bfc)[:, :1]
        total = jnp.squeeze(jnp.mean(atomwise))
        return total, atomwise

    (total, atomwise), d_r16 = jax.value_and_grad(energy, has_aux=True)(r16)
    pairwise_forces = -d_r16
    forces = _scatter_any(pairwise_forces, dst, N_NODES)[:, :3] * float(
        N_NODES)
    return total, forces, atomwise


# speedup vs baseline: 1.0113x; 1.0113x over previous
import functools

import jax
import jax.numpy as jnp
import numpy as np
from jax import lax
from jax.experimental import pallas as pl
from jax.experimental.pallas import tpu as pltpu
from jax.experimental.pallas import tpu_sc as plsc

N_NODES = 10000
ATOM_IN = 92
EDGE_BINS = 80
TRIPLET_BINS = 40
EMBED = 64
HIDDEN = 128
R_ONSET = 7.5
R_CUTOFF = 8.0

LANES = 128
BN = 1024


def _mm_fwd_kernel(x_ref, w_ref, b_ref, o_ref):
    o_ref[...] = (
        jnp.dot(x_ref[...], w_ref[...], preferred_element_type=jnp.float32)
        + b_ref[...]
    )


def _mm_nt_kernel(g_ref, w_ref, o_ref):
    o_ref[...] = lax.dot_general(
        g_ref[...], w_ref[...], (((1,), (1,)), ((), ())),
        preferred_element_type=jnp.float32)


def _mm_tn_kernel(x_ref, g_ref, dw_ref, db_ref, *, nrows):
    i = pl.program_id(0)
    rows = lax.broadcasted_iota(jnp.int32, (BN, 1), 0) + i * BN
    keep = rows < nrows
    x = jnp.where(keep, x_ref[...], 0.0)
    g = jnp.where(keep, g_ref[...], 0.0)

    @pl.when(i == 0)
    def _():
        dw_ref[...] = jnp.zeros_like(dw_ref)
        db_ref[...] = jnp.zeros_like(db_ref)

    dw_ref[...] += lax.dot_general(
        x, g, (((0,), (0,)), ((), ())), preferred_element_type=jnp.float32)
    db_ref[...] += jnp.sum(g, axis=0, keepdims=True)


def _mm_raw(x, w, b):
    n = x.shape[0]
    return pl.pallas_call(
        _mm_fwd_kernel,
        grid=(pl.cdiv(n, BN),),
        in_specs=[
            pl.BlockSpec((BN, LANES), lambda i: (i, 0)),
            pl.BlockSpec((LANES, LANES), lambda i: (0, 0)),
            pl.BlockSpec((1, LANES), lambda i: (0, 0)),
        ],
        out_specs=pl.BlockSpec((BN, LANES), lambda i: (i, 0)),
        out_shape=jax.ShapeDtypeStruct((n, LANES), jnp.float32),
    )(x, w, b.reshape(1, LANES))


def _mm_nt_raw(g, w):
    n = g.shape[0]
    return pl.pallas_call(
        _mm_nt_kernel,
        grid=(pl.cdiv(n, BN),),
        in_specs=[
            pl.BlockSpec((BN, LANES), lambda i: (i, 0)),
            pl.BlockSpec((LANES, LANES), lambda i: (0, 0)),
        ],
        out_specs=pl.BlockSpec((BN, LANES), lambda i: (i, 0)),
        out_shape=jax.ShapeDtypeStruct((n, LANES), jnp.float32),
    )(g, w)


def _mm_tn_raw(x, g):
    n = x.shape[0]
    dw, db = pl.pallas_call(
        functools.partial(_mm_tn_kernel, nrows=n),
        grid=(pl.cdiv(n, BN),),
        in_specs=[
            pl.BlockSpec((BN, LANES), lambda i: (i, 0)),
            pl.BlockSpec((BN, LANES), lambda i: (i, 0)),
        ],
        out_specs=[
            pl.BlockSpec((LANES, LANES), lambda i: (0, 0)),
            pl.BlockSpec((1, LANES), lambda i: (0, 0)),
        ],
        out_shape=[
            jax.ShapeDtypeStruct((LANES, LANES), jnp.float32),
            jax.ShapeDtypeStruct((1, LANES), jnp.float32),
        ],
    )(x, g)
    return dw, db.reshape(LANES)


@jax.custom_vjp
def mm(x, w, b):
    return _mm_raw(x, w, b)


def _mm_vfwd(x, w, b):
    return _mm_raw(x, w, b), (x, w)


def _mm_vbwd(res, g):
    x, w = res
    dx = _mm_nt_raw(g, w)
    dw, db = _mm_tn_raw(x, g)
    return dx, dw, db


mm.defvjp(_mm_vfwd, _mm_vbwd)


CH = 128
NW = 32


POOL_WORDS = 2040000


def _sc_gather_raw(table, idx):
    t_rows, d = table.shape
    b = idx.shape[0]
    per = b // NW
    nfull, tail = per // CH, per % CH
    GG = 4 if d >= 64 else 8
    ngrp, nrem = nfull // GG, nfull % GG
    mesh = plsc.VectorSubcoreMesh(core_axis_name="c", subcore_axis_name="s")

    @functools.partial(
        pl.kernel,
        out_type=jax.ShapeDtypeStruct((b, d), jnp.float32),
        mesh=mesh,
        compiler_params=pltpu.CompilerParams(use_tc_tiling_on_sc=False),
        scratch_types=[
            pltpu.VMEM((GG, CH), jnp.int32),
            pltpu.VMEM((GG, CH, d), jnp.float32),
            pltpu.VMEM((max(tail, 8),), jnp.int32),
            pltpu.VMEM((max(tail, 8), d), jnp.float32),
            pltpu.SemaphoreType.DMA,
            pltpu.SemaphoreType.DMA,
            pltpu.SemaphoreType.DMA,
        ],
    )
    def k(table_hbm, idx_hbm, out_hbm, idx_v, rows_v, idxt_v, rowst_v,
          sem_i, sem_g, sem_o):
        w = lax.axis_index("s") * 2 + lax.axis_index("c")
        base = w * per

        def group(g, nch):
            off0 = base + g * (GG * CH)
            hs = [pltpu.async_copy(
                idx_hbm.at[pl.ds(off0 + q * CH, CH)], idx_v.at[q], sem_i)
                for q in range(nch)]
            for h in hs:
                h.wait()
            hs = [pltpu.async_copy(
                table_hbm.at[idx_v.at[q]], rows_v.at[q], sem_g)
                for q in range(nch)]
            for h in hs:
                h.wait()
            hs = [pltpu.async_copy(
                rows_v.at[q], out_hbm.at[pl.ds(off0 + q * CH, CH)], sem_o)
                for q in range(nch)]
            for h in hs:
                h.wait()

        def body(g, _):
            group(g, GG)
            return 0

        lax.fori_loop(0, ngrp, body, 0)
        if nrem:
            group(ngrp, nrem)
        if tail:
            off = base + nfull * CH
            pltpu.sync_copy(idx_hbm.at[pl.ds(off, tail)], idxt_v)
            pltpu.async_copy(table_hbm.at[idxt_v], rowst_v, sem_g).wait()
            pltpu.sync_copy(rowst_v, out_hbm.at[pl.ds(off, tail)])

    return k(table, idx)


def _scatter_gg(dc):
    return 1 if dc > 64 else 8


def _scatter_dc(v_pad, d):
    gg = _scatter_gg(d)
    zr = 64 if d >= 128 else 128
    pertile = gg * CH * d + gg * CH + 2 * CH * d + 2 * CH + zr * d
    if v_pad * d + 16 * pertile <= POOL_WORDS:
        return d
    return 16


def _sc_scatter_raw(vals, idx, v_pad):
    b, d = vals.shape
    per = b // NW
    nfull, tail = per // CH, per % CH
    dc = _scatter_dc(v_pad, d)
    GG = _scatter_gg(dc)
    ngrp, nrem = nfull // GG, nfull % GG
    nd = d // dc
    rpt = v_pad // 16
    zr = min(rpt, 64 if dc >= 128 else 128)
    nz, ztail = rpt // zr, rpt % zr
    mesh = plsc.VectorSubcoreMesh(core_axis_name="c", subcore_axis_name="s")

    @functools.partial(
        pl.kernel,
        out_type=jax.ShapeDtypeStruct((2, v_pad, d), jnp.float32),
        mesh=mesh,
        compiler_params=pltpu.CompilerParams(
            use_tc_tiling_on_sc=False, has_side_effects=True),
        scratch_types=[
            pltpu.VMEM((GG, CH), jnp.int32),
            pltpu.VMEM((GG, CH, dc), jnp.float32),
            pltpu.VMEM((max(tail, 8),), jnp.int32),
            pltpu.VMEM((max(tail, 8), dc), jnp.float32),
            pltpu.VMEM((zr, dc), jnp.float32),
            pltpu.VMEM_SHARED((v_pad, dc), jnp.float32),
            pltpu.SemaphoreType.DMA,
            pltpu.SemaphoreType.DMA,
            pltpu.SemaphoreType.DMA,
        ],
    )
    def k(vals_hbm, idx_hbm, zeros_hbm, out_hbm, idx_v, vals_v, idxt_v,
          valst_v, z_v, shared, sem_i, sem_v, sem_a):
        c = lax.axis_index("c")
        s = lax.axis_index("s")
        w = s * 2 + c
        row0 = s * rpt
        base = w * per
        pltpu.sync_copy(zeros_hbm, z_v)

        def group(g, nch, c0):
            off0 = base + g * (GG * CH)
            hs = [pltpu.async_copy(
                idx_hbm.at[pl.ds(off0 + q * CH, CH)], idx_v.at[q], sem_i)
                for q in range(nch)]
            hs += [pltpu.async_copy(
                vals_hbm.at[pl.ds(off0 + q * CH, CH), pl.ds(c0, dc)],
                vals_v.at[q], sem_v) for q in range(nch)]
            for h in hs:
                h.wait()
            hs = [pltpu.async_copy(
                vals_v.at[q], shared.at[idx_v.at[q]], sem_a, add=True)
                for q in range(nch)]
            for h in hs:
                h.wait()

        for di in range(nd):
            c0 = di * dc

            def zbody(j, _):
                pltpu.sync_copy(z_v, shared.at[pl.ds(row0 + j * zr, zr)])
                return 0

            lax.fori_loop(0, nz, zbody, 0)
            if ztail:
                pltpu.sync_copy(z_v.at[pl.ds(0, ztail)],
                                shared.at[pl.ds(row0 + nz * zr, ztail)])
            plsc.subcore_barrier()

            def body(g, _):
                group(g, GG, c0)
                return 0

            lax.fori_loop(0, ngrp, body, 0)
            if nrem:
                group(ngrp, nrem, c0)
            if tail:
                off = base + nfull * CH
                pltpu.sync_copy(idx_hbm.at[pl.ds(off, tail)], idxt_v)
                pltpu.sync_copy(
                    vals_hbm.at[pl.ds(off, tail), pl.ds(c0, dc)], valst_v)
                pltpu.sync_copy(valst_v, shared.at[idxt_v], add=True)
            plsc.subcore_barrier()
            pltpu.sync_copy(
                shared.at[pl.ds(row0, rpt)],
                out_hbm.at[c, pl.ds(row0, rpt), pl.ds(c0, dc)])
            if nd > 1 and di < nd - 1:
                plsc.subcore_barrier()

    return k(vals, idx, jnp.zeros((zr, dc), jnp.float32))


def _round_up(n, m):
    return ((n + m - 1) // m) * m


def _pad_rows(x, n_pad, value=0.0):
    n = x.shape[0]
    if n == n_pad:
        return x
    return jnp.pad(x, ((0, n_pad - n),) + ((0, 0),) * (x.ndim - 1),
                   constant_values=value)


def _scatter_any(vals, idx, v_rows):
    d = vals.shape[1]
    cap = 102400
    pieces = []
    for lo in range(0, v_rows, cap):
        rows = min(cap, v_rows - lo)
        v_pad = _round_up(rows + 1, 128)
        local = idx - lo
        ok = (local >= 0) & (local < rows)
        local = jnp.where(ok, local, rows).astype(jnp.int32)
        part = _sc_scatter_raw(vals, local, v_pad)
        pieces.append((part[0] + part[1])[:rows])
    return jnp.concatenate(pieces, 0) if len(pieces) > 1 else pieces[0]


def _gather_any(table, idx):
    ok = (idx >= 0) & (idx < table.shape[0])
    safe = jnp.where(ok, idx, 0).astype(jnp.int32)
    out = _sc_gather_raw(table, safe)
    return out * ok[:, None].astype(jnp.float32)


@functools.partial(jax.custom_vjp, nondiff_argnums=())
def sc_gather(table, idx):
    return _gather_any(table, idx)


def _sc_gather_vfwd(table, idx):
    return _gather_any(table, idx), (idx, table.shape[0])


def _sc_gather_vbwd(res, g):
    idx, t_rows = res
    return _scatter_any(g, idx, t_rows), None


sc_gather.defvjp(_sc_gather_vfwd, _sc_gather_vbwd)


@jax.custom_vjp
def sc_gather_pair(table, idx, bwd_idx):
    return _gather_any(table, idx)


def _sc_gather_pair_vfwd(table, idx, bwd_idx):
    return _gather_any(table, idx), (bwd_idx,)


def _sc_gather_pair_vbwd(res, g):
    (bwd_idx,) = res
    return _gather_any(g, bwd_idx), None, None


sc_gather_pair.defvjp(_sc_gather_pair_vfwd, _sc_gather_pair_vbwd)


@functools.partial(jax.custom_vjp, nondiff_argnums=(2,))
def sc_segment_sum(vals, idx, v_rows):
    return _scatter_any(vals, idx, v_rows)


def _sc_segsum_vfwd(vals, idx, v_rows):
    return _scatter_any(vals, idx, v_rows), (idx,)


def _sc_segsum_vbwd(v_rows, res, g):
    (idx,) = res
    return _gather_any(g, idx), None


sc_segment_sum.defvjp(_sc_segsum_vfwd, _sc_segsum_vbwd)


def _pad_lin(p, din, dout):
    w = jnp.pad(p["W"], ((0, LANES - din), (0, LANES - dout)))
    b = jnp.pad(p["b"], (0, LANES - dout))
    return w, b


def _silu(x):
    return x * jax.nn.sigmoid(x)


def _masked_ln_silu(u, g, be, dtrue):
    if dtrue == LANES:
        mu = jnp.mean(u, -1, keepdims=True)
        dev = u - mu
        var = jnp.mean(dev * dev, -1, keepdims=True)
        return _silu(dev * lax.rsqrt(var + 1e-5) * g + be)
    colmask = (jnp.arange(LANES) < dtrue).astype(jnp.float32)
    mu = jnp.sum(u, -1, keepdims=True) / dtrue
    dev = (u - mu) * colmask
    var = jnp.sum(dev * dev, -1, keepdims=True) / dtrue
    return _silu(dev * lax.rsqrt(var + 1e-5) * g + be) * colmask


def _mlp(pp, x, dtrue):
    w, b, g, be = pp
    return _masked_ln_silu(mm(x, w, b), g, be, dtrue)


def _prep_mlp(p, din, dout):
    w, b = _pad_lin(p["lin"], din, dout)
    g = jnp.pad(p["g"], (0, LANES - dout))
    be = jnp.pad(p["be"], (0, LANES - dout))
    return (w, b, g, be)


def _rbf(d, vmin, vmax, bins):
    centers = jnp.linspace(vmin, vmax, bins)
    gamma = 1.0 / (centers[1] - centers[0])
    centers = jnp.concatenate(
        [centers, jnp.full((LANES - bins,), 1e9, jnp.float32)])
    return jnp.exp(-gamma * (d[:, None] - centers[None, :]) ** 2)


def _smooth_cutoff(r):
    rc2, ro2, r2 = R_CUTOFF**2, R_ONSET**2, r**2
    fc = ((rc2 - r2) ** 2 * (rc2 + 2.0 * r2 - 3.0 * ro2)) / (rc2 - ro2) ** 3
    return jnp.where(r < R_ONSET, 1.0, jnp.where(r > R_CUTOFF, 0.0, fc))


def _egc(p, src_i, dst_i, x, y, v_rows, cutoff=None):
    wsg, bsg = _pad_lin(p["src_gate"], HIDDEN, HIDDEN)
    wdg, bdg = _pad_lin(p["dst_gate"], HIDDEN, HIDDEN)
    weg, beg = _pad_lin(p["edge_gate"], HIDDEN, HIDDEN)
    wsu, bsu = _pad_lin(p["src_update"], HIDDEN, HIDDEN)
    wdu, bdu = _pad_lin(p["dst_update"], HIDDEN, HIDDEN)
    gxs = mm(x, wsg, bsg)
    gxd = mm(x, wdg, bdg)
    ey = mm(y, weg, beg)
    m = sc_gather(gxs, src_i) + sc_gather(gxd, dst_i) + ey
    sigma = jax.nn.sigmoid(m)
    if cutoff is not None:
        sigma = sigma * cutoff[:, None]
    bh = mm(x, wdu, bdu)
    sbh = sc_gather(bh, src_i)
    num = sc_segment_sum(sigma * sbh, dst_i, v_rows)
    den = sc_segment_sum(sigma, dst_i, v_rows)
    h = num / (den + 1e-6)
    x_out = x + _masked_ln_silu(mm(x, wsu, bsu) + h, p["ng"], p["nb"], HIDDEN)
    y_out = y + _masked_ln_silu(m, p["eg"], p["eb"], HIDDEN)
    return x_out, y_out


def kernel(atom_features, r, edge_index, y_mask, lg_src, lg_dst, params):
    src = edge_index[0]
    dst = edge_index[1]
    n_edges = src.shape[0]
    n_local = y_mask.shape[0]
    n_lg = lg_src.shape[0]

    n_local_p = _round_up(n_local + 1, 256)
    n_lg_p = _round_up(n_lg, 256)

    y_mask_p = _pad_rows(y_mask, n_local_p, -1)
    lg_src_p = _pad_rows(lg_src, n_lg_p, -1)
    lg_dst_p = _pad_rows(lg_dst, n_lg_p, -1)

    inv = jnp.full((n_edges,), -1, jnp.int32)
    inv = inv.at[y_mask].set(jnp.arange(n_local, dtype=jnp.int32))

    af_p = jnp.pad(atom_features, ((0, 0), (0, LANES - ATOM_IN)))
    r16 = jnp.pad(r, ((0, 0), (0, 13)))

    edge_mlp1 = _prep_mlp(params["edge_mlp1"], EDGE_BINS, EMBED)
    edge_mlp2 = _prep_mlp(params["edge_mlp2"], EMBED, HIDDEN)
    angle_mlp1 = _prep_mlp(params["angle_mlp1"], TRIPLET_BINS, EMBED)
    angle_mlp2 = _prep_mlp(params["angle_mlp2"], EMBED, HIDDEN)
    atom_mlp = _prep_mlp(params["atom_mlp"], ATOM_IN, HIDDEN)
    wfc, bfc = _pad_lin(params["fc"], HIDDEN, 1)

    def energy(r16_in):
        bondlength = jnp.sqrt(jnp.sum(r16_in * r16_in, 1))
        fcut = _smooth_cutoff(bondlength)
        y = _mlp(edge_mlp2,
                 _mlp(edge_mlp1, _rbf(bondlength, 0.0, 8.0, EDGE_BINS),
                      EMBED), HIDDEN)
        r_local = sc_gather_pair(r16_in, y_mask_p, inv)
        r1 = -sc_gather(r_local, lg_src_p)
        r2 = sc_gather(r_local, lg_dst_p)
        dotp = jnp.sum(r1 * r2, 1)
        nrm = jnp.sqrt(jnp.sum(r1 * r1, 1) * jnp.sum(r2 * r2, 1))
        cos = jnp.clip(dotp / jnp.maximum(nrm, 1e-30), -1.0, 1.0)
        z = _mlp(angle_mlp2,
                 _mlp(angle_mlp1, _rbf(cos, -1.0, 1.0, TRIPLET_BINS),
                      EMBED), HIDDEN)
        x = _mlp(atom_mlp, af_p, HIDDEN)
        for lp in params["alignn"]:
            ylocal = sc_gather_pair(y, y_mask_p, inv)
            m_e, z = _egc(lp["edge_update"], lg_src_p, lg_dst_p, ylocal, z,
                          n_local_p)
            y = y + sc_gather_pair(m_e - ylocal, inv, y_mask_p)
            x, y = _egc(lp["node_update"], src, dst, x, y, N_NODES, fcut)
        for lp in params["gcn"]:
            x, y = _egc(lp, src, dst, x, y, N_NODES, fcut)
        atomwise = mm(x, wfc, bfc)[:, :1]
        total = jnp.squeeze(jnp.mean(atomwise))
        return total, atomwise

    (total, atomwise), d_r16 = jax.value_and_grad(energy, has_aux=True)(r16)
    pairwise_forces = -d_r16
    forces = _scatter_any(pairwise_forces, dst, N_NODES)[:, :3] * float(
        N_NODES)
    return total, forces, atomwise


# speedup vs baseline: 1.0579x; 1.0461x over previous
import functools

import jax
import jax.numpy as jnp
import numpy as np
from jax import lax
from jax.experimental import pallas as pl
from jax.experimental.pallas import tpu as pltpu
from jax.experimental.pallas import tpu_sc as plsc

N_NODES = 10000
ATOM_IN = 92
EDGE_BINS = 80
TRIPLET_BINS = 40
EMBED = 64
HIDDEN = 128
R_ONSET = 7.5
R_CUTOFF = 8.0

LANES = 128
BN = 1024


def _mm_fwd_kernel(x_ref, w_ref, b_ref, o_ref):
    o_ref[...] = (
        jnp.dot(x_ref[...], w_ref[...], preferred_element_type=jnp.float32)
        + b_ref[...]
    )


def _mm_nt_kernel(g_ref, w_ref, o_ref):
    o_ref[...] = lax.dot_general(
        g_ref[...], w_ref[...], (((1,), (1,)), ((), ())),
        preferred_element_type=jnp.float32)


def _mm_tn_kernel(x_ref, g_ref, dw_ref, db_ref, *, nrows):
    i = pl.program_id(0)
    rows = lax.broadcasted_iota(jnp.int32, (BN, 1), 0) + i * BN
    keep = rows < nrows
    x = jnp.where(keep, x_ref[...], 0.0)
    g = jnp.where(keep, g_ref[...], 0.0)

    @pl.when(i == 0)
    def _():
        dw_ref[...] = jnp.zeros_like(dw_ref)
        db_ref[...] = jnp.zeros_like(db_ref)

    dw_ref[...] += lax.dot_general(
        x, g, (((0,), (0,)), ((), ())), preferred_element_type=jnp.float32)
    db_ref[...] += jnp.sum(g, axis=0, keepdims=True)


def _mm_raw(x, w, b):
    n = x.shape[0]
    return pl.pallas_call(
        _mm_fwd_kernel,
        grid=(pl.cdiv(n, BN),),
        in_specs=[
            pl.BlockSpec((BN, LANES), lambda i: (i, 0)),
            pl.BlockSpec((LANES, LANES), lambda i: (0, 0)),
            pl.BlockSpec((1, LANES), lambda i: (0, 0)),
        ],
        out_specs=pl.BlockSpec((BN, LANES), lambda i: (i, 0)),
        out_shape=jax.ShapeDtypeStruct((n, LANES), jnp.float32),
    )(x, w, b.reshape(1, LANES))


def _mm_nt_raw(g, w):
    n = g.shape[0]
    return pl.pallas_call(
        _mm_nt_kernel,
        grid=(pl.cdiv(n, BN),),
        in_specs=[
            pl.BlockSpec((BN, LANES), lambda i: (i, 0)),
            pl.BlockSpec((LANES, LANES), lambda i: (0, 0)),
        ],
        out_specs=pl.BlockSpec((BN, LANES), lambda i: (i, 0)),
        out_shape=jax.ShapeDtypeStruct((n, LANES), jnp.float32),
    )(g, w)


def _mm_tn_raw(x, g):
    n = x.shape[0]
    dw, db = pl.pallas_call(
        functools.partial(_mm_tn_kernel, nrows=n),
        grid=(pl.cdiv(n, BN),),
        in_specs=[
            pl.BlockSpec((BN, LANES), lambda i: (i, 0)),
            pl.BlockSpec((BN, LANES), lambda i: (i, 0)),
        ],
        out_specs=[
            pl.BlockSpec((LANES, LANES), lambda i: (0, 0)),
            pl.BlockSpec((1, LANES), lambda i: (0, 0)),
        ],
        out_shape=[
            jax.ShapeDtypeStruct((LANES, LANES), jnp.float32),
            jax.ShapeDtypeStruct((1, LANES), jnp.float32),
        ],
    )(x, g)
    return dw, db.reshape(LANES)


@jax.custom_vjp
def mm(x, w, b):
    return _mm_raw(x, w, b)


def _mm_vfwd(x, w, b):
    return _mm_raw(x, w, b), (x, w)


def _mm_vbwd(res, g):
    x, w = res
    dx = _mm_nt_raw(g, w)
    dw, db = _mm_tn_raw(x, g)
    return dx, dw, db


mm.defvjp(_mm_vfwd, _mm_vbwd)


CH = 128
NW = 32


POOL_WORDS = 2040000


def _sc_gather_raw(table, idx):
    t_rows, d = table.shape
    b = idx.shape[0]
    per = b // NW
    nfull, tail = per // CH, per % CH
    GG = 4 if d >= 64 else 8
    ngrp, nrem = nfull // GG, nfull % GG
    mesh = plsc.VectorSubcoreMesh(core_axis_name="c", subcore_axis_name="s")

    @functools.partial(
        pl.kernel,
        out_type=jax.ShapeDtypeStruct((b, d), jnp.float32),
        mesh=mesh,
        compiler_params=pltpu.CompilerParams(use_tc_tiling_on_sc=False),
        scratch_types=[
            pltpu.VMEM((GG, CH), jnp.int32),
            pltpu.VMEM((GG, CH, d), jnp.float32),
            pltpu.VMEM((max(tail, 8),), jnp.int32),
            pltpu.VMEM((max(tail, 8), d), jnp.float32),
            pltpu.SemaphoreType.DMA,
            pltpu.SemaphoreType.DMA,
            pltpu.SemaphoreType.DMA,
        ],
    )
    def k(table_hbm, idx_hbm, out_hbm, idx_v, rows_v, idxt_v, rowst_v,
          sem_i, sem_g, sem_o):
        w = lax.axis_index("s") * 2 + lax.axis_index("c")
        base = w * per

        def group(g, nch):
            off0 = base + g * (GG * CH)
            hs = [pltpu.async_copy(
                idx_hbm.at[pl.ds(off0 + q * CH, CH)], idx_v.at[q], sem_i)
                for q in range(nch)]
            for h in hs:
                h.wait()
            hs = [pltpu.async_copy(
                table_hbm.at[idx_v.at[q]], rows_v.at[q], sem_g)
                for q in range(nch)]
            for h in hs:
                h.wait()
            hs = [pltpu.async_copy(
                rows_v.at[q], out_hbm.at[pl.ds(off0 + q * CH, CH)], sem_o)
                for q in range(nch)]
            for h in hs:
                h.wait()

        def body(g, _):
            group(g, GG)
            return 0

        lax.fori_loop(0, ngrp, body, 0)
        if nrem:
            group(ngrp, nrem)
        if tail:
            off = base + nfull * CH
            pltpu.sync_copy(idx_hbm.at[pl.ds(off, tail)], idxt_v)
            pltpu.async_copy(table_hbm.at[idxt_v], rowst_v, sem_g).wait()
            pltpu.sync_copy(rowst_v, out_hbm.at[pl.ds(off, tail)])

    return k(table, idx)


def _scatter_gg(dc):
    return 4 if dc >= 64 else 8


def _scatter_dc(v_pad, d):
    for dc in (128, 64, 32, 16):
        if dc > d:
            continue
        gg = _scatter_gg(dc)
        pertile = gg * CH * dc + gg * CH + 2 * CH * dc + 2 * CH + 128 * dc
        if v_pad * dc + 16 * pertile <= POOL_WORDS:
            return dc
    return 16


def _sc_scatter_raw(vals, idx, v_pad):
    b, d = vals.shape
    per = b // NW
    nfull, tail = per // CH, per % CH
    dc = _scatter_dc(v_pad, d)
    GG = _scatter_gg(dc)
    ngrp, nrem = nfull // GG, nfull % GG
    nd = d // dc
    rpt = v_pad // 16
    zr = min(rpt, 128)
    nz, ztail = rpt // zr, rpt % zr
    mesh = plsc.VectorSubcoreMesh(core_axis_name="c", subcore_axis_name="s")

    @functools.partial(
        pl.kernel,
        out_type=jax.ShapeDtypeStruct((2, v_pad, d), jnp.float32),
        mesh=mesh,
        compiler_params=pltpu.CompilerParams(
            use_tc_tiling_on_sc=False, has_side_effects=True),
        scratch_types=[
            pltpu.VMEM((GG, CH), jnp.int32),
            pltpu.VMEM((GG, CH, dc), jnp.float32),
            pltpu.VMEM((max(tail, 8),), jnp.int32),
            pltpu.VMEM((max(tail, 8), dc), jnp.float32),
            pltpu.VMEM((zr, dc), jnp.float32),
            pltpu.VMEM_SHARED((v_pad, dc), jnp.float32),
            pltpu.SemaphoreType.DMA,
            pltpu.SemaphoreType.DMA,
            pltpu.SemaphoreType.DMA,
        ],
    )
    def k(vals_hbm, idx_hbm, zeros_hbm, out_hbm, idx_v, vals_v, idxt_v,
          valst_v, z_v, shared, sem_i, sem_v, sem_a):
        c = lax.axis_index("c")
        s = lax.axis_index("s")
        w = s * 2 + c
        row0 = s * rpt
        base = w * per
        pltpu.sync_copy(zeros_hbm, z_v)

        def group(g, nch, c0):
            off0 = base + g * (GG * CH)
            hs = [pltpu.async_copy(
                idx_hbm.at[pl.ds(off0 + q * CH, CH)], idx_v.at[q], sem_i)
                for q in range(nch)]
            hs += [pltpu.async_copy(
                vals_hbm.at[pl.ds(off0 + q * CH, CH), pl.ds(c0, dc)],
                vals_v.at[q], sem_v) for q in range(nch)]
            for h in hs:
                h.wait()
            hs = [pltpu.async_copy(
                vals_v.at[q], shared.at[idx_v.at[q]], sem_a, add=True)
                for q in range(nch)]
            for h in hs:
                h.wait()

        for di in range(nd):
            c0 = di * dc

            def zbody(j, _):
                pltpu.sync_copy(z_v, shared.at[pl.ds(row0 + j * zr, zr)])
                return 0

            lax.fori_loop(0, nz, zbody, 0)
            if ztail:
                pltpu.sync_copy(z_v.at[pl.ds(0, ztail)],
                                shared.at[pl.ds(row0 + nz * zr, ztail)])
            plsc.subcore_barrier()

            def body(g, _):
                group(g, GG, c0)
                return 0

            lax.fori_loop(0, ngrp, body, 0)
            if nrem:
                group(ngrp, nrem, c0)
            if tail:
                off = base + nfull * CH
                pltpu.sync_copy(idx_hbm.at[pl.ds(off, tail)], idxt_v)
                pltpu.sync_copy(
                    vals_hbm.at[pl.ds(off, tail), pl.ds(c0, dc)], valst_v)
                pltpu.sync_copy(valst_v, shared.at[idxt_v], add=True)
            plsc.subcore_barrier()
            pltpu.sync_copy(
                shared.at[pl.ds(row0, rpt)],
                out_hbm.at[c, pl.ds(row0, rpt), pl.ds(c0, dc)])
            if nd > 1 and di < nd - 1:
                plsc.subcore_barrier()

    return k(vals, idx, jnp.zeros((zr, dc), jnp.float32))


V_WIN = 12544


def _sc_scatter_sorted_raw(vals, s_ids, perm, blo, bn, npass):
    b, d = vals.shape
    v_pad = V_WIN + 128
    rpt = v_pad // 16
    cpt = V_WIN // 16
    zr = 64
    nz, ztail = rpt // zr, rpt % zr
    mesh = plsc.VectorSubcoreMesh(core_axis_name="c", subcore_axis_name="s")

    @functools.partial(
        pl.kernel,
        out_type=jax.ShapeDtypeStruct((2, npass * V_WIN, d), jnp.float32),
        mesh=mesh,
        compiler_params=pltpu.CompilerParams(
            use_tc_tiling_on_sc=False, has_side_effects=True,
            needs_layout_passes=False),
        scratch_types=[
            pltpu.VMEM((CH,), jnp.int32),
            pltpu.VMEM((CH,), jnp.int32),
            pltpu.VMEM((CH,), jnp.int32),
            pltpu.VMEM((CH, d), jnp.float32),
            pltpu.VMEM((zr, d), jnp.float32),
            pltpu.VMEM((16,), jnp.int32),
            pltpu.VMEM((16,), jnp.int32),
            pltpu.VMEM_SHARED((v_pad, d), jnp.float32),
            pltpu.SemaphoreType.DMA,
        ],
    )
    def k(vals_hbm, sids_hbm, perm_hbm, blo_hbm, bn_hbm, zeros_hbm, out_hbm,
          pidx_v, idx_v, idx2_v, vals_v, z_v, blo_v, bn_v, shared, sem_g):
        c = lax.axis_index("c")
        s = lax.axis_index("s")
        w = s * 2 + c
        row0 = s * rpt
        crow0 = s * cpt
        pltpu.sync_copy(zeros_hbm, z_v)
        pltpu.sync_copy(blo_hbm, blo_v)
        pltpu.sync_copy(bn_hbm, bn_v)
        lanes = lax.broadcasted_iota(jnp.int32, (16,), 0)
        blov = blo_v[...]
        bnv = bn_v[...]
        for p in range(npass):
            lo = jnp.sum(jnp.where(lanes == p, blov, 0))
            nch = jnp.sum(jnp.where(lanes == p, bnv, 0))

            def zbody(j, _):
                pltpu.sync_copy(z_v, shared.at[pl.ds(row0 + j * zr, zr)])
                return 0

            lax.fori_loop(0, nz, zbody, 0)
            if ztail:
                pltpu.sync_copy(z_v.at[pl.ds(0, ztail)],
                                shared.at[pl.ds(row0 + nz * zr, ztail)])
            plsc.subcore_barrier()
            myn = lax.max(0, (nch - w + 31) // 32)
            base = p * V_WIN

            def chunk(jj, _):
                off = pl.multiple_of(lo + (w + jj * 32) * CH, CH)
                pltpu.sync_copy(perm_hbm.at[pl.ds(off, CH)], pidx_v)
                pltpu.async_copy(vals_hbm.at[pidx_v], vals_v, sem_g).wait()
                pltpu.sync_copy(sids_hbm.at[pl.ds(off, CH)], idx_v)
                for g in range(CH // 16):
                    v = idx_v[pl.ds(g * 16, 16)]
                    local = v - base
                    ok = (local >= 0) & (local < V_WIN)
                    idx2_v[pl.ds(g * 16, 16)] = jnp.where(ok, local, V_WIN)
                pltpu.sync_copy(vals_v, shared.at[idx2_v], add=True)
                return 0

            lax.fori_loop(0, myn, chunk, 0)
            plsc.subcore_barrier()
            pltpu.sync_copy(
                shared.at[pl.ds(crow0, cpt)],
                out_hbm.at[c, pl.ds(p * V_WIN + crow0, cpt)])
            if p < npass - 1:
                plsc.subcore_barrier()

    return k(vals, s_ids, perm, blo, bn, jnp.zeros((zr, d), jnp.float32))


def _sorted_scatter_call(npass, vals, s_ids, perm, blo, bn):
    parts = _sc_scatter_sorted_raw(vals, s_ids, perm, blo, bn, npass)
    return parts[0] + parts[1]


@functools.partial(jax.custom_vjp, nondiff_argnums=(0,))
def _gather_sortb(npass, table, idx_sent, s_ids, perm, blo, bn):
    return _gather_any(table, idx_sent)


def _gather_sortb_fwd(npass, table, idx_sent, s_ids, perm, blo, bn):
    res = (table.shape[0], idx_sent, s_ids, perm, blo, bn)
    return _gather_any(table, idx_sent), res


def _gather_sortb_bwd(npass, res, g):
    t_rows, idx_sent, s_ids, perm, blo, bn = res
    dt = _sorted_scatter_call(npass, g, s_ids, perm, blo, bn)[:t_rows]
    return dt, None, None, None, None, None


_gather_sortb.defvjp(_gather_sortb_fwd, _gather_sortb_bwd)


@functools.partial(jax.custom_vjp, nondiff_argnums=(0, 1))
def _segsum_sortf(npass, v_rows, vals, ids_sent, s_ids, perm, blo, bn):
    return _sorted_scatter_call(npass, vals, s_ids, perm, blo, bn)[:v_rows]


def _segsum_sortf_fwd(npass, v_rows, vals, ids_sent, s_ids, perm, blo, bn):
    out = _sorted_scatter_call(npass, vals, s_ids, perm, blo, bn)[:v_rows]
    return out, (ids_sent,)


def _segsum_sortf_bwd(npass, v_rows, res, g):
    (ids_sent,) = res
    return _gather_any(g, ids_sent), None, None, None, None, None


_segsum_sortf.defvjp(_segsum_sortf_fwd, _segsum_sortf_bwd)


def _round_up(n, m):
    return ((n + m - 1) // m) * m


def _pad_rows(x, n_pad, value=0.0):
    n = x.shape[0]
    if n == n_pad:
        return x
    return jnp.pad(x, ((0, n_pad - n),) + ((0, 0),) * (x.ndim - 1),
                   constant_values=value)


def _scatter_any(vals, idx, v_rows):
    d = vals.shape[1]
    cap = 102400
    pieces = []
    for lo in range(0, v_rows, cap):
        rows = min(cap, v_rows - lo)
        v_pad = _round_up(rows + 1, 128)
        local = idx - lo
        ok = (local >= 0) & (local < rows)
        local = jnp.where(ok, local, rows).astype(jnp.int32)
        part = _sc_scatter_raw(vals, local, v_pad)
        pieces.append((part[0] + part[1])[:rows])
    return jnp.concatenate(pieces, 0) if len(pieces) > 1 else pieces[0]


def _gather_any(table, idx):
    ok = (idx >= 0) & (idx < table.shape[0])
    safe = jnp.where(ok, idx, 0).astype(jnp.int32)
    out = _sc_gather_raw(table, safe)
    return out * ok[:, None].astype(jnp.float32)


@functools.partial(jax.custom_vjp, nondiff_argnums=())
def sc_gather(table, idx):
    return _gather_any(table, idx)


def _sc_gather_vfwd(table, idx):
    return _gather_any(table, idx), (idx, table.shape[0])


def _sc_gather_vbwd(res, g):
    idx, t_rows = res
    return _scatter_any(g, idx, t_rows), None


sc_gather.defvjp(_sc_gather_vfwd, _sc_gather_vbwd)


@jax.custom_vjp
def sc_gather_pair(table, idx, bwd_idx):
    return _gather_any(table, idx)


def _sc_gather_pair_vfwd(table, idx, bwd_idx):
    return _gather_any(table, idx), (bwd_idx,)


def _sc_gather_pair_vbwd(res, g):
    (bwd_idx,) = res
    return _gather_any(g, bwd_idx), None, None


sc_gather_pair.defvjp(_sc_gather_pair_vfwd, _sc_gather_pair_vbwd)


@functools.partial(jax.custom_vjp, nondiff_argnums=(2,))
def sc_segment_sum(vals, idx, v_rows):
    return _scatter_any(vals, idx, v_rows)


def _sc_segsum_vfwd(vals, idx, v_rows):
    return _scatter_any(vals, idx, v_rows), (idx,)


def _sc_segsum_vbwd(v_rows, res, g):
    (idx,) = res
    return _gather_any(g, idx), None


sc_segment_sum.defvjp(_sc_segsum_vfwd, _sc_segsum_vbwd)


def _pad_lin(p, din, dout):
    w = jnp.pad(p["W"], ((0, LANES - din), (0, LANES - dout)))
    b = jnp.pad(p["b"], (0, LANES - dout))
    return w, b


def _silu(x):
    return x * jax.nn.sigmoid(x)


def _masked_ln_silu(u, g, be, dtrue):
    if dtrue == LANES:
        mu = jnp.mean(u, -1, keepdims=True)
        dev = u - mu
        var = jnp.mean(dev * dev, -1, keepdims=True)
        return _silu(dev * lax.rsqrt(var + 1e-5) * g + be)
    colmask = (jnp.arange(LANES) < dtrue).astype(jnp.float32)
    mu = jnp.sum(u, -1, keepdims=True) / dtrue
    dev = (u - mu) * colmask
    var = jnp.sum(dev * dev, -1, keepdims=True) / dtrue
    return _silu(dev * lax.rsqrt(var + 1e-5) * g + be) * colmask


def _mlp(pp, x, dtrue):
    w, b, g, be = pp
    return _masked_ln_silu(mm(x, w, b), g, be, dtrue)


def _prep_mlp(p, din, dout):
    w, b = _pad_lin(p["lin"], din, dout)
    g = jnp.pad(p["g"], (0, LANES - dout))
    be = jnp.pad(p["be"], (0, LANES - dout))
    return (w, b, g, be)


def _rbf(d, vmin, vmax, bins):
    centers = jnp.linspace(vmin, vmax, bins)
    gamma = 1.0 / (centers[1] - centers[0])
    centers = jnp.concatenate(
        [centers, jnp.full((LANES - bins,), 1e9, jnp.float32)])
    return jnp.exp(-gamma * (d[:, None] - centers[None, :]) ** 2)


def _smooth_cutoff(r):
    rc2, ro2, r2 = R_CUTOFF**2, R_ONSET**2, r**2
    fc = ((rc2 - r2) ** 2 * (rc2 + 2.0 * r2 - 3.0 * ro2)) / (rc2 - ro2) ** 3
    return jnp.where(r < R_ONSET, 1.0, jnp.where(r > R_CUTOFF, 0.0, fc))


def _egc(p, src_i, dst_i, x, y, v_rows, cutoff=None, lgctx=None):
    wsg, bsg = _pad_lin(p["src_gate"], HIDDEN, HIDDEN)
    wdg, bdg = _pad_lin(p["dst_gate"], HIDDEN, HIDDEN)
    weg, beg = _pad_lin(p["edge_gate"], HIDDEN, HIDDEN)
    wsu, bsu = _pad_lin(p["src_update"], HIDDEN, HIDDEN)
    wdu, bdu = _pad_lin(p["dst_update"], HIDDEN, HIDDEN)
    gxs = mm(x, wsg, bsg)
    gxd = mm(x, wdg, bdg)
    ey = mm(y, weg, beg)
    if lgctx is None:
        gs = sc_gather(gxs, src_i)
        gd = sc_gather(gxd, dst_i)
    else:
        npass, sctx, dctx = lgctx
        gs = _gather_sortb(npass, gxs, src_i, *sctx)
        gd = _gather_sortb(npass, gxd, dst_i, *dctx)
    m = gs + gd + ey
    sigma = jax.nn.sigmoid(m)
    if cutoff is not None:
        sigma = sigma * cutoff[:, None]
    bh = mm(x, wdu, bdu)
    if lgctx is None:
        sbh = sc_gather(bh, src_i)
        num = sc_segment_sum(sigma * sbh, dst_i, v_rows)
        den = sc_segment_sum(sigma, dst_i, v_rows)
    else:
        sbh = _gather_sortb(npass, bh, src_i, *sctx)
        num = _segsum_sortf(npass, v_rows, sigma * sbh, dst_i, *dctx)
        den = _segsum_sortf(npass, v_rows, sigma, dst_i, *dctx)
    h = num / (den + 1e-6)
    x_out = x + _masked_ln_silu(mm(x, wsu, bsu) + h, p["ng"], p["nb"], HIDDEN)
    y_out = y + _masked_ln_silu(m, p["eg"], p["eb"], HIDDEN)
    return x_out, y_out


def kernel(atom_features, r, edge_index, y_mask, lg_src, lg_dst, params):
    src = edge_index[0]
    dst = edge_index[1]
    n_edges = src.shape[0]
    n_local = y_mask.shape[0]
    n_lg = lg_src.shape[0]

    n_local_p = _round_up(n_local + 1, 256)
    n_lg_p = _round_up(n_lg, 256)

    y_mask_p = _pad_rows(y_mask, n_local_p, -1)
    lg_src_p = _pad_rows(lg_src, n_lg_p, -1)
    lg_dst_p = _pad_rows(lg_dst, n_lg_p, -1)

    inv = jnp.full((n_edges,), -1, jnp.int32)
    inv = inv.at[y_mask].set(jnp.arange(n_local, dtype=jnp.int32))

    af_p = jnp.pad(atom_features, ((0, 0), (0, LANES - ATOM_IN)))
    r16 = jnp.pad(r, ((0, 0), (0, 13)))

    def _make_sorted_ctx(idx_sent, presorted):
        key = jnp.where(idx_sent < 0, jnp.int32(1 << 30), idx_sent)
        if presorted:
            perm = jnp.arange(idx_sent.shape[0], dtype=jnp.int32)
            s_ids = key
        else:
            perm = jnp.argsort(key).astype(jnp.int32)
            s_ids = key[perm]
        lg_npass = -(-n_local_p // V_WIN)
        bounds = (jnp.arange(lg_npass + 1) * V_WIN).astype(jnp.int32)
        cuts = jnp.searchsorted(s_ids, bounds).astype(jnp.int32)
        lo = (cuts[:-1] // CH) * CH
        hi = ((cuts[1:] + CH - 1) // CH) * CH
        nch = (hi - lo) // CH
        blo = jnp.zeros(16, jnp.int32).at[:lg_npass].set(lo)
        bn = jnp.zeros(16, jnp.int32).at[:lg_npass].set(nch)
        return (s_ids, perm, blo, bn)

    lg_npass = -(-n_local_p // V_WIN)
    lgctx = (lg_npass, _make_sorted_ctx(lg_src_p, False),
             _make_sorted_ctx(lg_dst_p, True))

    edge_mlp1 = _prep_mlp(params["edge_mlp1"], EDGE_BINS, EMBED)
    edge_mlp2 = _prep_mlp(params["edge_mlp2"], EMBED, HIDDEN)
    angle_mlp1 = _prep_mlp(params["angle_mlp1"], TRIPLET_BINS, EMBED)
    angle_mlp2 = _prep_mlp(params["angle_mlp2"], EMBED, HIDDEN)
    atom_mlp = _prep_mlp(params["atom_mlp"], ATOM_IN, HIDDEN)
    wfc, bfc = _pad_lin(params["fc"], HIDDEN, 1)

    def energy(r16_in):
        bondlength = jnp.sqrt(jnp.sum(r16_in * r16_in, 1))
        fcut = _smooth_cutoff(bondlength)
        y = _mlp(edge_mlp2,
                 _mlp(edge_mlp1, _rbf(bondlength, 0.0, 8.0, EDGE_BINS),
                      EMBED), HIDDEN)
        r_local = sc_gather_pair(r16_in, y_mask_p, inv)
        r1 = -sc_gather(r_local, lg_src_p)
        r2 = sc_gather(r_local, lg_dst_p)
        dotp = jnp.sum(r1 * r2, 1)
        nrm = jnp.sqrt(jnp.sum(r1 * r1, 1) * jnp.sum(r2 * r2, 1))
        cos = jnp.clip(dotp / jnp.maximum(nrm, 1e-30), -1.0, 1.0)
        z = _mlp(angle_mlp2,
                 _mlp(angle_mlp1, _rbf(cos, -1.0, 1.0, TRIPLET_BINS),
                      EMBED), HIDDEN)
        x = _mlp(atom_mlp, af_p, HIDDEN)
        for lp in params["alignn"]:
            ylocal = sc_gather_pair(y, y_mask_p, inv)
            m_e, z = _egc(lp["edge_update"], lg_src_p, lg_dst_p, ylocal, z,
                          n_local_p, lgctx=lgctx)
            y = y + sc_gather_pair(m_e - ylocal, inv, y_mask_p)
            x, y = _egc(lp["node_update"], src, dst, x, y, N_NODES, fcut)
        for lp in params["gcn"]:
            x, y = _egc(lp, src, dst, x, y, N_NODES, fcut)
        atomwise = mm(x, wfc, bfc)[:, :1]
        total = jnp.squeeze(jnp.mean(atomwise))
        return total, atomwise

    (total, atomwise), d_r16 = jax.value_and_grad(energy, has_aux=True)(r16)
    pairwise_forces = -d_r16
    forces = _scatter_any(pairwise_forces, dst, N_NODES)[:, :3] * float(
        N_NODES)
    return total, forces, atomwise
